# Initial kernel scaffold; baseline (speedup 1.0000x reference)
import jax
import jax.numpy as jnp
from jax.experimental import pallas as pl


def kernel(user_feature, edge_index, edge_type, G_features, adj_indices, adj_values, mu, n_id, params, num_cluster_iter):
    raise NotImplementedError("write your pallas kernel here")


# speedup vs baseline: 1.6476x; 1.6476x over previous
import functools

import jax
import jax.numpy as jnp
from jax.experimental import pallas as pl

_BLK = 1000


def _dense_body(nx, has_bias, act, *refs):
    xs = refs[:nx]
    ws = refs[nx:2 * nx]
    i = 2 * nx
    acc = xs[0][...] @ ws[0][...]
    for j in range(1, nx):
        acc = acc + xs[j][...] @ ws[j][...]
    if has_bias:
        acc = acc + refs[i][...]
        i += 1
    if act == "relu":
        acc = jnp.maximum(acc, 0.0)
    elif act == "sigmoid":
        acc = jax.nn.sigmoid(acc)
    refs[-1][...] = acc


def _dense(xs, Ws, b=None, act=None):
    n = xs[0].shape[0]
    O = Ws[0].shape[1]
    nx = len(xs)
    in_specs = [pl.BlockSpec((_BLK, x.shape[1]), lambda i: (i, 0)) for x in xs]
    in_specs += [pl.BlockSpec(W.shape, lambda i: (0, 0)) for W in Ws]
    args = list(xs) + list(Ws)
    if b is not None:
        b2 = b.reshape(1, O)
        in_specs.append(pl.BlockSpec((1, O), lambda i: (0, 0)))
        args.append(b2)
    return pl.pallas_call(
        functools.partial(_dense_body, nx, b is not None, act),
        grid=(n // _BLK,),
        in_specs=in_specs,
        out_specs=pl.BlockSpec((_BLK, O), lambda i: (i, 0)),
        out_shape=jax.ShapeDtypeStruct((n, O), jnp.float32),
    )(*args)


def kernel(user_feature, edge_index, edge_type, G_features, adj_indices,
           adj_values, mu, n_id, params, num_cluster_iter):
    N = user_feature.shape[0]
    x_ae = user_feature[:, 8:]
    cat_in = user_feature[:, :3]
    num_in = user_feature[:, 3:8]

    e0 = _dense([x_ae], [params['enc_in'][0]], params['enc_in'][1], "relu")
    e1 = _dense([e0], [params['hid_enc'][0][0]], params['hid_enc'][0][1], "relu")

    row, col = adj_indices[0], adj_indices[1]

    def spmm(x):
        return jax.ops.segment_sum(adj_values[:, None] * jnp.take(x, col, axis=0),
                                   row, num_segments=N)

    g1 = _dense([G_features], [params['gc1'][0]])
    h = jax.nn.relu(spmm(g1) + params['gc1'][1])
    h2 = _dense([h], [params['gc2'][0]])
    embeds = spmm(h2) + params['gc2'][1]
    emb_g = jnp.take(embeds, n_id, axis=0)

    c = _dense([cat_in], [params['cat'][0]], params['cat'][1])
    nm = _dense([num_in], [params['num'][0]], params['num'][1])
    tw = _dense([x_ae], [params['tweet'][0]], params['tweet'][1])
    Wf, bf = params['feat']
    f0 = _dense([c, nm, tw], [Wf[:128], Wf[128:256], Wf[256:]], bf)

    src, dst = edge_index[0], edge_index[1]
    gidx = edge_type * N + src
    sidx = edge_type * N + dst
    cnt = jax.ops.segment_sum(jnp.ones((src.shape[0],), jnp.float32), sidx,
                              num_segments=2 * N)
    inv = (1.0 / jnp.maximum(cnt, 1.0)).reshape(2, N, 1)

    def rgcn(xs, p):
        Wrel, Wroot, b = p
        off = [0]
        for x in xs:
            off.append(off[-1] + x.shape[1])
        t0 = _dense(xs, [Wrel[0, off[j]:off[j + 1]] for j in range(len(xs))])
        t1 = _dense(xs, [Wrel[1, off[j]:off[j + 1]] for j in range(len(xs))])
        root = _dense(xs, [Wroot[off[j]:off[j + 1]] for j in range(len(xs))], b)
        tcat = jnp.concatenate([t0, t1], axis=0)
        acc = jax.ops.segment_sum(jnp.take(tcat, gidx, axis=0), sidx,
                                  num_segments=2 * N)
        return root + (acc.reshape(2, N, -1) * inv).sum(0)

    f1 = rgcn([f0], params['gnn_in'])
    f2 = rgcn([f1, e0, emb_g], params['hid_gnn'][0])
    f3 = rgcn([f2, e1, emb_g], params['hid_gnn'][1])
    f4 = rgcn([f3], params['gnn_out'])
    f5 = rgcn([f4], params['gnn_comm'])
    feat = jax.nn.sigmoid(f5)

    h1 = _dense([feat], [params['bd1'][0]], params['bd1'][1], "relu")
    logits = _dense([h1], [params['bd2'][0]], params['bd2'][1])
    return jax.nn.softmax(logits, axis=1)


# speedup vs baseline: 5.7113x; 3.4665x over previous
import functools

import jax
import jax.numpy as jnp
from jax import lax
from jax.experimental import pallas as pl
from jax.experimental.pallas import tpu as pltpu
from jax.experimental.pallas import tpu_sc as plsc

_N = 10000
_E = 320000
_BLK = 1000

_CPT1 = 160
_EPAD1 = _CPT1 * 16 * 128
_A1 = 20480
_CPT2 = 80
_EPAD2 = _CPT2 * 32 * 128
_A2 = 10240
_NPAD = 12288

def _mesh():
    return plsc.VectorSubcoreMesh(core_axis_name="c", subcore_axis_name="s")


def _dense_body(nx, has_bias, act, *refs):
    xs = refs[:nx]
    ws = refs[nx:2 * nx]
    i = 2 * nx
    acc = xs[0][...] @ ws[0][...]
    for j in range(1, nx):
        acc = acc + xs[j][...] @ ws[j][...]
    if has_bias:
        acc = acc + refs[i][...]
        i += 1
    if act == "relu":
        acc = jnp.maximum(acc, 0.0)
    elif act == "sigmoid":
        acc = jax.nn.sigmoid(acc)
    refs[-1][...] = acc


def _dense(xs, Ws, b=None, act=None):
    n = xs[0].shape[0]
    O = Ws[0].shape[1]
    nx = len(xs)
    in_specs = [pl.BlockSpec((_BLK, x.shape[1]), lambda i: (i, 0)) for x in xs]
    in_specs += [pl.BlockSpec(W.shape, lambda i: (0, 0)) for W in Ws]
    args = list(xs) + list(Ws)
    if b is not None:
        in_specs.append(pl.BlockSpec((1, O), lambda i: (0, 0)))
        args.append(b.reshape(1, O))
    return pl.pallas_call(
        functools.partial(_dense_body, nx, b is not None, act),
        grid=(n // _BLK,),
        in_specs=in_specs,
        out_specs=pl.BlockSpec((_BLK, O), lambda i: (i, 0)),
        out_shape=jax.ShapeDtypeStruct((n, O), jnp.float32),
    )(*args)


@functools.lru_cache(maxsize=None)
def _rgcn_agg(dh):
    out_t = (jax.ShapeDtypeStruct((_A1, dh), jnp.float32),) * 2
    scratch = [
        pltpu.VMEM((_CPT1, 128), jnp.int32),
        pltpu.VMEM((_CPT1, 128), jnp.int32),
        pltpu.VMEM((128, dh), jnp.float32),
        pltpu.VMEM_SHARED((_A1, dh), jnp.float32),
        pltpu.SemaphoreType.DMA,
    ]

    @functools.partial(pl.kernel, mesh=_mesh(), out_type=out_t,
                       scratch_types=scratch,
                       compiler_params=pltpu.CompilerParams(
                           use_tc_tiling_on_sc=False))
    def k(t0, t1, gidx, sidx, zeros, o0, o1, idx_v, sdx_v, rows_v, acc, sem):
        c = lax.axis_index("c")
        s = lax.axis_index("s")
        zr = _A1 // 16
        pltpu.sync_copy(zeros, acc.at[pl.ds(s * zr, zr)])
        pltpu.sync_copy(gidx.at[pl.ds(s * _CPT1, _CPT1)], idx_v)
        pltpu.sync_copy(sidx.at[pl.ds(s * _CPT1, _CPT1)], sdx_v)
        plsc.subcore_barrier()

        def run(tbl):
            def body(j, carry):
                pltpu.async_copy(tbl.at[idx_v.at[j]], rows_v, sem).wait()
                pltpu.sync_copy(rows_v, acc.at[sdx_v.at[j]], add=True)
                return carry
            lax.fori_loop(0, _CPT1, body, 0)

        @pl.when(c == 0)
        def _():
            run(t0)

        @pl.when(c == 1)
        def _():
            run(t1)

        plsc.subcore_barrier()
        wb = _A1 // 16

        @pl.when(c == 0)
        def _():
            pltpu.sync_copy(acc.at[pl.ds(s * wb, wb)], o0.at[pl.ds(s * wb, wb)])

        @pl.when(c == 1)
        def _():
            pltpu.sync_copy(acc.at[pl.ds(s * wb, wb)], o1.at[pl.ds(s * wb, wb)])

    return k


@functools.lru_cache(maxsize=None)
def _spmm_agg(d):
    out_t = (jax.ShapeDtypeStruct((_A2, d), jnp.float32),) * 2
    scratch = [
        pltpu.VMEM((_CPT2, 128), jnp.int32),
        pltpu.VMEM((_CPT2, 128), jnp.int32),
        pltpu.VMEM((_CPT2, 128), jnp.float32),
        pltpu.VMEM((128, d), jnp.float32),
        pltpu.VMEM_SHARED((_A2, d), jnp.float32),
        pltpu.SemaphoreType.DMA,
    ]
    ncg = d // 16

    @functools.partial(pl.kernel, mesh=_mesh(), out_type=out_t,
                       scratch_types=scratch,
                       compiler_params=pltpu.CompilerParams(
                           use_tc_tiling_on_sc=False))
    def k(tbl, gidx, sidx, w, zeros, o0, o1, idx_v, sdx_v, w_v, rows_v, acc,
          sem):
        c = lax.axis_index("c")
        s = lax.axis_index("s")
        zr = _A2 // 16
        pltpu.sync_copy(zeros, acc.at[pl.ds(s * zr, zr)])
        base = (c * 16 + s) * _CPT2
        pltpu.sync_copy(gidx.at[pl.ds(base, _CPT2)], idx_v)
        pltpu.sync_copy(sidx.at[pl.ds(base, _CPT2)], sdx_v)
        pltpu.sync_copy(w.at[pl.ds(base, _CPT2)], w_v)
        plsc.subcore_barrier()

        def body(j, carry):
            pltpu.async_copy(tbl.at[idx_v.at[j]], rows_v, sem).wait()

            def mgrp(g, cc):
                wvec = w_v[j, pl.ds(g * 16, 16)]
                for i in range(16):
                    r = g * 16 + i
                    wv = wvec[i]
                    for kk in range(ncg):
                        sl = pl.ds(kk * 16, 16)
                        rows_v[r, sl] = rows_v[r, sl] * wv
                return cc
            lax.fori_loop(0, 8, mgrp, 0)
            pltpu.sync_copy(rows_v, acc.at[sdx_v.at[j]], add=True)
            return carry
        lax.fori_loop(0, _CPT2, body, 0)

        plsc.subcore_barrier()
        wb = _A2 // 16

        @pl.when(c == 0)
        def _():
            pltpu.sync_copy(acc.at[pl.ds(s * wb, wb)], o0.at[pl.ds(s * wb, wb)])

        @pl.when(c == 1)
        def _():
            pltpu.sync_copy(acc.at[pl.ds(s * wb, wb)], o1.at[pl.ds(s * wb, wb)])

    return k


@functools.lru_cache(maxsize=None)
def _cnt_kernel():
    out_t = (jax.ShapeDtypeStruct((_A1, 16), jnp.float32),) * 2
    scratch = [
        pltpu.VMEM((_CPT2, 128), jnp.int32),
        pltpu.VMEM((128, 16), jnp.float32),
        pltpu.VMEM_SHARED((_A1, 16), jnp.float32),
    ]

    @functools.partial(pl.kernel, mesh=_mesh(), out_type=out_t,
                       scratch_types=scratch,
                       compiler_params=pltpu.CompilerParams(
                           use_tc_tiling_on_sc=False))
    def k(sidx, ones, zeros, o0, o1, sdx_v, ones_v, acc):
        c = lax.axis_index("c")
        s = lax.axis_index("s")
        zr = _A1 // 16
        pltpu.sync_copy(zeros, acc.at[pl.ds(s * zr, zr)])
        base = (c * 16 + s) * _CPT2
        pltpu.sync_copy(sidx.at[pl.ds(base, _CPT2)], sdx_v)
        pltpu.sync_copy(ones, ones_v)
        plsc.subcore_barrier()

        def body(j, carry):
            pltpu.sync_copy(ones_v, acc.at[sdx_v.at[j]], add=True)
            return carry
        lax.fori_loop(0, _CPT2, body, 0)

        plsc.subcore_barrier()
        wb = _A1 // 16

        @pl.when(c == 0)
        def _():
            pltpu.sync_copy(acc.at[pl.ds(s * wb, wb)], o0.at[pl.ds(s * wb, wb)])

        @pl.when(c == 1)
        def _():
            pltpu.sync_copy(acc.at[pl.ds(s * wb, wb)], o1.at[pl.ds(s * wb, wb)])

    return k


@functools.lru_cache(maxsize=None)
def _emb_gather():
    out_t = jax.ShapeDtypeStruct((_NPAD, 32), jnp.float32)
    scratch = [
        pltpu.VMEM((3, 128), jnp.int32),
        pltpu.VMEM((128, 32), jnp.float32),
        pltpu.SemaphoreType.DMA,
    ]

    @functools.partial(pl.kernel, mesh=_mesh(), out_type=out_t,
                       scratch_types=scratch,
                       compiler_params=pltpu.CompilerParams(
                           use_tc_tiling_on_sc=False))
    def k(tbl, gidx, out, idx_v, rows_v, sem):
        c = lax.axis_index("c")
        s = lax.axis_index("s")
        w = c * 16 + s
        pltpu.sync_copy(gidx.at[w], idx_v)
        for j in range(3):
            pltpu.async_copy(tbl.at[idx_v.at[j]], rows_v, sem).wait()
            pltpu.sync_copy(rows_v, out.at[pl.ds((w * 3 + j) * 128, 128)])

    return k


def _pad_reshape(x, total, fill):
    return jnp.concatenate(
        [x, jnp.full((total - x.shape[0],), fill, x.dtype)]).reshape(-1, 128)


def kernel(user_feature, edge_index, edge_type, G_features, adj_indices,
           adj_values, mu, n_id, params, num_cluster_iter):
    N = _N
    x_ae = user_feature[:, 8:]
    cat_in = user_feature[:, :3]
    num_in = user_feature[:, 3:8]

    e0 = _dense([x_ae], [params['enc_in'][0]], params['enc_in'][1], "relu")
    e1 = _dense([e0], [params['hid_enc'][0][0]], params['hid_enc'][0][1], "relu")

    src, dst = edge_index[0], edge_index[1]
    et = edge_type.astype(jnp.int32)
    gidx1 = _pad_reshape(et * N + src, _EPAD1, 0)
    sidx1 = _pad_reshape(et * N + dst, _EPAD1, 2 * N)
    row, col = adj_indices[0], adj_indices[1]
    gidx2 = _pad_reshape(col, _EPAD2, 0)
    sidx2 = _pad_reshape(row, _EPAD2, N)
    wpad = _pad_reshape(adj_values, _EPAD2, 0.0)
    csidx = _pad_reshape(et * N + dst, _EPAD2, 2 * N)
    eidx = _pad_reshape(n_id.astype(jnp.int32), _NPAD, 0).reshape(32, 3, 128)

    zeros_a1 = {dh: jnp.zeros((_A1 // 16, dh), jnp.float32) for dh in (64, 32)}
    zeros_a2 = {d: jnp.zeros((_A2 // 16, d), jnp.float32) for d in (64, 32)}
    zeros_c = jnp.zeros((_A1 // 16, 16), jnp.float32)
    ones_c = jnp.ones((128, 16), jnp.float32)

    c0, c1 = _cnt_kernel()(csidx, ones_c, zeros_c)
    cnt = (c0 + c1)[:2 * N, 0]
    inv = (1.0 / jnp.maximum(cnt, 1.0)).reshape(2, N, 1)

    def spmm(x):
        d = x.shape[1]
        p0, p1 = _spmm_agg(d)(x, gidx2, sidx2, wpad, zeros_a2[d])
        return (p0 + p1)[:_N]

    g1 = _dense([G_features], [params['gc1'][0]])
    h = jax.nn.relu(spmm(g1) + params['gc1'][1])
    h2 = _dense([h], [params['gc2'][0]])
    embeds = spmm(h2) + params['gc2'][1]
    emb_g = _emb_gather()(embeds, eidx)[:N]

    c = _dense([cat_in], [params['cat'][0]], params['cat'][1])
    nm = _dense([num_in], [params['num'][0]], params['num'][1])
    tw = _dense([x_ae], [params['tweet'][0]], params['tweet'][1])
    Wf, bf = params['feat']
    f0 = _dense([c, nm, tw], [Wf[:128], Wf[128:256], Wf[256:]], bf)

    def rgcn(xs, p):
        Wrel, Wroot, b = p
        off = [0]
        for x in xs:
            off.append(off[-1] + x.shape[1])
        t0 = _dense(xs, [Wrel[0, off[j]:off[j + 1]] for j in range(len(xs))])
        t1 = _dense(xs, [Wrel[1, off[j]:off[j + 1]] for j in range(len(xs))])
        root = _dense(xs, [Wroot[off[j]:off[j + 1]] for j in range(len(xs))], b)
        O = t0.shape[1]
        dh = O // 2
        tcat = jnp.concatenate([t0, t1], axis=0)
        o0, o1 = _rgcn_agg(dh)(tcat[:, :dh], tcat[:, dh:], gidx1, sidx1,
                               zeros_a1[dh])
        acc = jnp.concatenate([o0[:2 * N], o1[:2 * N]], axis=1).reshape(2, N, O)
        return root + (acc * inv).sum(0)

    f1 = rgcn([f0], params['gnn_in'])
    f2 = rgcn([f1, e0, emb_g], params['hid_gnn'][0])
    f3 = rgcn([f2, e1, emb_g], params['hid_gnn'][1])
    f4 = rgcn([f3], params['gnn_out'])
    f5 = rgcn([f4], params['gnn_comm'])
    feat = jax.nn.sigmoid(f5)

    h1 = _dense([feat], [params['bd1'][0]], params['bd1'][1], "relu")
    logits = _dense([h1], [params['bd2'][0]], params['bd2'][1])
    return jax.nn.softmax(logits, axis=1)


# speedup vs baseline: 7.0108x; 1.2275x over previous
import functools

import jax
import jax.numpy as jnp
from jax import lax
from jax.experimental import pallas as pl
from jax.experimental.pallas import tpu as pltpu
from jax.experimental.pallas import tpu_sc as plsc

_N = 10000
_E = 320000
_BLK = 1000

_CPT1 = 160
_EPAD1 = _CPT1 * 16 * 128
_A1 = 20160
_CPT2 = 80
_EPAD2 = _CPT2 * 32 * 128
_A2 = 10080
_CPTS = 160
_NPAD = 12288

def _mesh():
    return plsc.VectorSubcoreMesh(core_axis_name="c", subcore_axis_name="s")


def _dense_body(nx, has_bias, act, *refs):
    xs = refs[:nx]
    ws = refs[nx:2 * nx]
    i = 2 * nx
    acc = xs[0][...] @ ws[0][...]
    for j in range(1, nx):
        acc = acc + xs[j][...] @ ws[j][...]
    if has_bias:
        acc = acc + refs[i][...]
        i += 1
    if act == "relu":
        acc = jnp.maximum(acc, 0.0)
    elif act == "sigmoid":
        acc = jax.nn.sigmoid(acc)
    refs[-1][...] = acc


def _dense(xs, Ws, b=None, act=None):
    n = xs[0].shape[0]
    O = Ws[0].shape[1]
    nx = len(xs)
    in_specs = [pl.BlockSpec((_BLK, x.shape[1]), lambda i: (i, 0)) for x in xs]
    in_specs += [pl.BlockSpec(W.shape, lambda i: (0, 0)) for W in Ws]
    args = list(xs) + list(Ws)
    if b is not None:
        in_specs.append(pl.BlockSpec((1, O), lambda i: (0, 0)))
        args.append(b.reshape(1, O))
    return pl.pallas_call(
        functools.partial(_dense_body, nx, b is not None, act),
        grid=(n // _BLK,),
        in_specs=in_specs,
        out_specs=pl.BlockSpec((_BLK, O), lambda i: (i, 0)),
        out_shape=jax.ShapeDtypeStruct((n, O), jnp.float32),
    )(*args)


def _agg_run(tbl, idx_v, sdx_v, rows_v, acc, gsa, gsb, ssa, ssb, cpt):
    K = 4
    ng2 = cpt // (2 * K)
    dummy = tbl.at[pl.ds(0, 128)]

    def fire(go, bufo, sem):
        for b in range(K):
            pltpu.async_copy(tbl.at[idx_v.at[go + b]], rows_v.at[bufo + b],
                             sem)

    def drain(sem):
        for b in range(K):
            pltpu.make_async_copy(dummy, rows_v.at[b], sem).wait()

    def fire_scat(go, bufo, sem):
        for b in range(K):
            pltpu.async_copy(rows_v.at[bufo + b], acc.at[sdx_v.at[go + b]],
                             sem, add=True)

    fire(0, 0, gsa)

    def body(g2, carry):
        g0 = 2 * g2 * K
        fire(g0 + K, K, gsb)
        drain(gsa)
        fire_scat(g0, 0, ssa)
        drain(ssa)

        @pl.when(g2 < ng2 - 1)
        def _():
            fire(g0 + 2 * K, 0, gsa)

        drain(gsb)
        fire_scat(g0 + K, K, ssb)
        drain(ssb)
        return carry
    lax.fori_loop(0, ng2, body, 0)


@functools.lru_cache(maxsize=None)
def _rgcn_half():
    out_t = (jax.ShapeDtypeStruct((_A1, 32), jnp.float32),) * 2
    scratch = [
        pltpu.VMEM((_CPT1, 128), jnp.int32),
        pltpu.VMEM((_CPT1, 128), jnp.int32),
        pltpu.VMEM((8, 128, 32), jnp.float32),
        pltpu.VMEM_SHARED((_A1, 32), jnp.float32),
        pltpu.SemaphoreType.DMA,
        pltpu.SemaphoreType.DMA,
        pltpu.SemaphoreType.DMA,
        pltpu.SemaphoreType.DMA,
    ]

    @functools.partial(pl.kernel, mesh=_mesh(), out_type=out_t,
                       scratch_types=scratch,
                       compiler_params=pltpu.CompilerParams(
                           use_tc_tiling_on_sc=False))
    def k(t0, t1, gidx, sidx, zeros, o0, o1, idx_v, sdx_v, rows_v, acc,
          gsa, gsb, ssa, ssb):
        c = lax.axis_index("c")
        s = lax.axis_index("s")
        zr = _A1 // 16
        pltpu.sync_copy(zeros, acc.at[pl.ds(s * zr, zr)])
        pltpu.sync_copy(gidx.at[pl.ds(s * _CPT1, _CPT1)], idx_v)
        pltpu.sync_copy(sidx.at[pl.ds(s * _CPT1, _CPT1)], sdx_v)
        plsc.subcore_barrier()

        @pl.when(c == 0)
        def _():
            _agg_run(t0, idx_v, sdx_v, rows_v, acc, gsa, gsb, ssa, ssb, _CPT1)

        @pl.when(c == 1)
        def _():
            _agg_run(t1, idx_v, sdx_v, rows_v, acc, gsa, gsb, ssa, ssb, _CPT1)

        plsc.subcore_barrier()

        @pl.when(c == 0)
        def _():
            pltpu.sync_copy(acc.at[pl.ds(s * zr, zr)], o0.at[pl.ds(s * zr, zr)])

        @pl.when(c == 1)
        def _():
            pltpu.sync_copy(acc.at[pl.ds(s * zr, zr)], o1.at[pl.ds(s * zr, zr)])

    return k


@functools.lru_cache(maxsize=None)
def _rgcn_quad():
    out_t = (jax.ShapeDtypeStruct((_A1, 32), jnp.float32),) * 4
    scratch = [
        pltpu.VMEM((_CPT1, 128), jnp.int32),
        pltpu.VMEM((_CPT1, 128), jnp.int32),
        pltpu.VMEM((8, 128, 32), jnp.float32),
        pltpu.VMEM_SHARED((_A1, 32), jnp.float32),
        pltpu.SemaphoreType.DMA,
        pltpu.SemaphoreType.DMA,
        pltpu.SemaphoreType.DMA,
        pltpu.SemaphoreType.DMA,
    ]

    @functools.partial(pl.kernel, mesh=_mesh(), out_type=out_t,
                       scratch_types=scratch,
                       compiler_params=pltpu.CompilerParams(
                           use_tc_tiling_on_sc=False))
    def k(q0, q1, q2, q3, gidx, sidx, zeros, o0, o1, o2, o3,
          idx_v, sdx_v, rows_v, acc, gsa, gsb, ssa, ssb):
        c = lax.axis_index("c")
        s = lax.axis_index("s")
        zr = _A1 // 16
        rows = pl.ds(s * zr, zr)
        pltpu.sync_copy(zeros, acc.at[rows])
        pltpu.sync_copy(gidx.at[pl.ds(s * _CPT1, _CPT1)], idx_v)
        pltpu.sync_copy(sidx.at[pl.ds(s * _CPT1, _CPT1)], sdx_v)
        plsc.subcore_barrier()

        @pl.when(c == 0)
        def _():
            _agg_run(q0, idx_v, sdx_v, rows_v, acc, gsa, gsb, ssa, ssb, _CPT1)

        @pl.when(c == 1)
        def _():
            _agg_run(q2, idx_v, sdx_v, rows_v, acc, gsa, gsb, ssa, ssb, _CPT1)

        plsc.subcore_barrier()

        @pl.when(c == 0)
        def _():
            pltpu.sync_copy(acc.at[rows], o0.at[rows])

        @pl.when(c == 1)
        def _():
            pltpu.sync_copy(acc.at[rows], o2.at[rows])

        pltpu.sync_copy(zeros, acc.at[rows])
        plsc.subcore_barrier()

        @pl.when(c == 0)
        def _():
            _agg_run(q1, idx_v, sdx_v, rows_v, acc, gsa, gsb, ssa, ssb, _CPT1)

        @pl.when(c == 1)
        def _():
            _agg_run(q3, idx_v, sdx_v, rows_v, acc, gsa, gsb, ssa, ssb, _CPT1)

        plsc.subcore_barrier()

        @pl.when(c == 0)
        def _():
            pltpu.sync_copy(acc.at[rows], o1.at[rows])

        @pl.when(c == 1)
        def _():
            pltpu.sync_copy(acc.at[rows], o3.at[rows])

    return k


@functools.lru_cache(maxsize=None)
def _spmm_agg(d):
    d2 = d // 2
    out_t = (jax.ShapeDtypeStruct((_A2, d2), jnp.float32),) * 2
    scratch = [
        pltpu.VMEM((_CPTS, 128), jnp.int32),
        pltpu.VMEM((_CPTS, 128), jnp.int32),
        pltpu.VMEM((_CPTS, 128), jnp.float32),
        pltpu.VMEM((8, 128, d2), jnp.float32),
        pltpu.VMEM_SHARED((_A2, d2), jnp.float32),
        pltpu.SemaphoreType.DMA,
        pltpu.SemaphoreType.DMA,
        pltpu.SemaphoreType.DMA,
        pltpu.SemaphoreType.DMA,
    ]
    ncg = d2 // 16

    @functools.partial(pl.kernel, mesh=_mesh(), out_type=out_t,
                       scratch_types=scratch,
                       compiler_params=pltpu.CompilerParams(
                           use_tc_tiling_on_sc=False))
    def k(t0, t1, gidx, sidx, w, zeros, o0, o1, idx_v, sdx_v, w_v, rows_v, acc,
          gsa, gsb, ssa, ssb):
        c = lax.axis_index("c")
        s = lax.axis_index("s")
        zr = _A2 // 16
        pltpu.sync_copy(zeros, acc.at[pl.ds(s * zr, zr)])
        pltpu.sync_copy(gidx.at[pl.ds(s * _CPTS, _CPTS)], idx_v)
        pltpu.sync_copy(sidx.at[pl.ds(s * _CPTS, _CPTS)], sdx_v)
        pltpu.sync_copy(w.at[pl.ds(s * _CPTS, _CPTS)], w_v)
        plsc.subcore_barrier()
        K = 4
        ng2 = _CPTS // (2 * K)

        def run(tbl):
            dummy = tbl.at[pl.ds(0, 128)]

            def fire(go, bufo, sem):
                for b in range(K):
                    pltpu.async_copy(tbl.at[idx_v.at[go + b]],
                                     rows_v.at[bufo + b], sem)

            def drain(sem):
                for b in range(K):
                    pltpu.make_async_copy(dummy, rows_v.at[b], sem).wait()

            def mul(go, bufo):
                for b in range(K):
                    def mgrp(g, cc, _b=b):
                        wvec = w_v[go + _b, pl.ds(g * 16, 16)]
                        for i in range(16):
                            r = g * 16 + i
                            wv = wvec[i]
                            for kk in range(ncg):
                                sl = pl.ds(kk * 16, 16)
                                rows_v[bufo + _b, r, sl] = (
                                    rows_v[bufo + _b, r, sl] * wv)
                        return cc
                    lax.fori_loop(0, 8, mgrp, 0)

            def fire_scat(go, bufo, sem):
                for b in range(K):
                    pltpu.async_copy(rows_v.at[bufo + b],
                                     acc.at[sdx_v.at[go + b]], sem, add=True)

            fire(0, 0, gsa)

            def body(g2, carry):
                g0 = 2 * g2 * K
                fire(g0 + K, K, gsb)
                drain(gsa)
                mul(g0, 0)
                fire_scat(g0, 0, ssa)
                drain(ssa)

                @pl.when(g2 < ng2 - 1)
                def _():
                    fire(g0 + 2 * K, 0, gsa)

                drain(gsb)
                mul(g0 + K, K)
                fire_scat(g0 + K, K, ssb)
                drain(ssb)
                return carry
            lax.fori_loop(0, ng2, body, 0)

        @pl.when(c == 0)
        def _():
            run(t0)

        @pl.when(c == 1)
        def _():
            run(t1)

        plsc.subcore_barrier()
        wb = _A2 // 16

        @pl.when(c == 0)
        def _():
            pltpu.sync_copy(acc.at[pl.ds(s * wb, wb)], o0.at[pl.ds(s * wb, wb)])

        @pl.when(c == 1)
        def _():
            pltpu.sync_copy(acc.at[pl.ds(s * wb, wb)], o1.at[pl.ds(s * wb, wb)])

    return k


@functools.lru_cache(maxsize=None)
def _emb_gather():
    out_t = jax.ShapeDtypeStruct((_NPAD, 32), jnp.float32)
    scratch = [
        pltpu.VMEM((3, 128), jnp.int32),
        pltpu.VMEM((128, 32), jnp.float32),
        pltpu.SemaphoreType.DMA,
    ]

    @functools.partial(pl.kernel, mesh=_mesh(), out_type=out_t,
                       scratch_types=scratch,
                       compiler_params=pltpu.CompilerParams(
                           use_tc_tiling_on_sc=False))
    def k(tbl, gidx, out, idx_v, rows_v, sem):
        c = lax.axis_index("c")
        s = lax.axis_index("s")
        w = c * 16 + s
        pltpu.sync_copy(gidx.at[w], idx_v)
        for j in range(3):
            pltpu.async_copy(tbl.at[idx_v.at[j]], rows_v, sem).wait()
            pltpu.sync_copy(rows_v, out.at[pl.ds((w * 3 + j) * 128, 128)])

    return k


def _pad_reshape(x, total, fill):
    return jnp.concatenate(
        [x, jnp.full((total - x.shape[0],), fill, x.dtype)]).reshape(-1, 128)


def kernel(user_feature, edge_index, edge_type, G_features, adj_indices,
           adj_values, mu, n_id, params, num_cluster_iter):
    N = _N
    x_ae = user_feature[:, 8:]
    cat_in = user_feature[:, :3]
    num_in = user_feature[:, 3:8]

    e0 = _dense([x_ae], [params['enc_in'][0]], params['enc_in'][1], "relu")
    e1 = _dense([e0], [params['hid_enc'][0][0]], params['hid_enc'][0][1], "relu")

    src, dst = edge_index[0], edge_index[1]
    et = edge_type.astype(jnp.int32)
    gidx1 = _pad_reshape(et * N + src, _EPAD1, 0)
    sidx1 = _pad_reshape(et * N + dst, _EPAD1, 2 * N)
    row, col = adj_indices[0], adj_indices[1]
    gidx2 = _pad_reshape(col, _EPAD2, 0)
    sidx2 = _pad_reshape(row, _EPAD2, N)
    wpad = _pad_reshape(adj_values, _EPAD2, 0.0)
    eidx = _pad_reshape(n_id.astype(jnp.int32), _NPAD, 0).reshape(32, 3, 128)

    zeros_a1 = jnp.zeros((_A1 // 16, 32), jnp.float32)
    zeros_a2 = {d: jnp.zeros((_A2 // 16, d // 2), jnp.float32) for d in (64, 32)}
    ones_t = jnp.ones((2 * N, 32), jnp.float32)

    c0, _unused = _rgcn_half()(ones_t, ones_t, gidx1, sidx1, zeros_a1)
    cnt = c0[:2 * N, 0]
    inv = (1.0 / jnp.maximum(cnt, 1.0)).reshape(2, N, 1)

    def spmm(x):
        d = x.shape[1]
        d2 = d // 2
        o0, o1 = _spmm_agg(d)(x[:, :d2], x[:, d2:], gidx2, sidx2, wpad,
                              zeros_a2[d])
        return jnp.concatenate([o0[:_N], o1[:_N]], axis=1)

    g1 = _dense([G_features], [params['gc1'][0]])
    h = jax.nn.relu(spmm(g1) + params['gc1'][1])
    h2 = _dense([h], [params['gc2'][0]])
    embeds = spmm(h2) + params['gc2'][1]
    emb_g = _emb_gather()(embeds, eidx)[:N]

    c = _dense([cat_in], [params['cat'][0]], params['cat'][1])
    nm = _dense([num_in], [params['num'][0]], params['num'][1])
    tw = _dense([x_ae], [params['tweet'][0]], params['tweet'][1])
    Wf, bf = params['feat']
    f0 = _dense([c, nm, tw], [Wf[:128], Wf[128:256], Wf[256:]], bf)

    def rgcn(xs, p):
        Wrel, Wroot, b = p
        off = [0]
        for x in xs:
            off.append(off[-1] + x.shape[1])
        t0 = _dense(xs, [Wrel[0, off[j]:off[j + 1]] for j in range(len(xs))])
        t1 = _dense(xs, [Wrel[1, off[j]:off[j + 1]] for j in range(len(xs))])
        root = _dense(xs, [Wroot[off[j]:off[j + 1]] for j in range(len(xs))], b)
        O = t0.shape[1]
        tcat = jnp.concatenate([t0, t1], axis=0)
        if O == 128:
            qs = [tcat[:, i * 32:(i + 1) * 32] for i in range(4)]
            os_ = _rgcn_quad()(qs[0], qs[1], qs[2], qs[3], gidx1, sidx1,
                               zeros_a1)
        else:
            os_ = _rgcn_half()(tcat[:, :32], tcat[:, 32:], gidx1, sidx1,
                               zeros_a1)
        acc = jnp.concatenate([o[:2 * N] for o in os_],
                              axis=1).reshape(2, N, O)
        return root + (acc * inv).sum(0)

    f1 = rgcn([f0], params['gnn_in'])
    f2 = rgcn([f1, e0, emb_g], params['hid_gnn'][0])
    f3 = rgcn([f2, e1, emb_g], params['hid_gnn'][1])
    f4 = rgcn([f3], params['gnn_out'])
    f5 = rgcn([f4], params['gnn_comm'])
    feat = jax.nn.sigmoid(f5)

    h1 = _dense([feat], [params['bd1'][0]], params['bd1'][1], "relu")
    logits = _dense([h1], [params['bd2'][0]], params['bd2'][1])
    return jax.nn.softmax(logits, axis=1)


# speedup vs baseline: 7.0112x; 1.0001x over previous
import functools

import jax
import jax.numpy as jnp
from jax import lax
from jax.experimental import pallas as pl
from jax.experimental.pallas import tpu as pltpu
from jax.experimental.pallas import tpu_sc as plsc

_N = 10000
_E = 320000
_BLK = 1000

_CPT1 = 160
_EPAD1 = _CPT1 * 16 * 128
_A1 = 20160
_CPT2 = 80
_EPAD2 = _CPT2 * 32 * 128
_A2 = 10080
_CPTS = 160
_NPAD = 12288

def _mesh():
    return plsc.VectorSubcoreMesh(core_axis_name="c", subcore_axis_name="s")


def _dense_body(nx, has_bias, act, *refs):
    xs = refs[:nx]
    ws = refs[nx:2 * nx]
    i = 2 * nx
    acc = xs[0][...] @ ws[0][...]
    for j in range(1, nx):
        acc = acc + xs[j][...] @ ws[j][...]
    if has_bias:
        acc = acc + refs[i][...]
        i += 1
    if act == "relu":
        acc = jnp.maximum(acc, 0.0)
    elif act == "sigmoid":
        acc = jax.nn.sigmoid(acc)
    refs[-1][...] = acc


def _dense(xs, Ws, b=None, act=None):
    n = xs[0].shape[0]
    O = Ws[0].shape[1]
    nx = len(xs)
    in_specs = [pl.BlockSpec((_BLK, x.shape[1]), lambda i: (i, 0)) for x in xs]
    in_specs += [pl.BlockSpec(W.shape, lambda i: (0, 0)) for W in Ws]
    args = list(xs) + list(Ws)
    if b is not None:
        in_specs.append(pl.BlockSpec((1, O), lambda i: (0, 0)))
        args.append(b.reshape(1, O))
    return pl.pallas_call(
        functools.partial(_dense_body, nx, b is not None, act),
        grid=(n // _BLK,),
        in_specs=in_specs,
        out_specs=pl.BlockSpec((_BLK, O), lambda i: (i, 0)),
        out_shape=jax.ShapeDtypeStruct((n, O), jnp.float32),
    )(*args)


def _agg_run(tbl, idx_v, sdx_v, rows_v, acc, gsa, gsb, ssa, ssb, cpt):
    K = 5
    ng2 = cpt // (2 * K)
    dummy = tbl.at[pl.ds(0, 128)]

    def fire(go, bufo, sem):
        for b in range(K):
            pltpu.async_copy(tbl.at[idx_v.at[go + b]], rows_v.at[bufo + b],
                             sem)

    def drain(sem):
        for b in range(K):
            pltpu.make_async_copy(dummy, rows_v.at[b], sem).wait()

    def fire_scat(go, bufo, sem):
        for b in range(K):
            pltpu.async_copy(rows_v.at[bufo + b], acc.at[sdx_v.at[go + b]],
                             sem, add=True)

    fire(0, 0, gsa)

    def body(g2, carry):
        g0 = 2 * g2 * K
        fire(g0 + K, K, gsb)
        drain(gsa)
        fire_scat(g0, 0, ssa)
        drain(ssa)

        @pl.when(g2 < ng2 - 1)
        def _():
            fire(g0 + 2 * K, 0, gsa)

        drain(gsb)
        fire_scat(g0 + K, K, ssb)
        drain(ssb)
        return carry
    lax.fori_loop(0, ng2, body, 0)


@functools.lru_cache(maxsize=None)
def _rgcn_half():
    out_t = (jax.ShapeDtypeStruct((_A1, 32), jnp.float32),) * 2
    scratch = [
        pltpu.VMEM((_CPT1, 128), jnp.int32),
        pltpu.VMEM((_CPT1, 128), jnp.int32),
        pltpu.VMEM((10, 128, 32), jnp.float32),
        pltpu.VMEM_SHARED((_A1, 32), jnp.float32),
        pltpu.SemaphoreType.DMA,
        pltpu.SemaphoreType.DMA,
        pltpu.SemaphoreType.DMA,
        pltpu.SemaphoreType.DMA,
    ]

    @functools.partial(pl.kernel, mesh=_mesh(), out_type=out_t,
                       scratch_types=scratch,
                       compiler_params=pltpu.CompilerParams(
                           use_tc_tiling_on_sc=False))
    def k(t0, t1, gidx, sidx, zeros, o0, o1, idx_v, sdx_v, rows_v, acc,
          gsa, gsb, ssa, ssb):
        c = lax.axis_index("c")
        s = lax.axis_index("s")
        zr = _A1 // 16
        pltpu.sync_copy(zeros, acc.at[pl.ds(s * zr, zr)])
        pltpu.sync_copy(gidx.at[pl.ds(s * _CPT1, _CPT1)], idx_v)
        pltpu.sync_copy(sidx.at[pl.ds(s * _CPT1, _CPT1)], sdx_v)
        plsc.subcore_barrier()

        @pl.when(c == 0)
        def _():
            _agg_run(t0, idx_v, sdx_v, rows_v, acc, gsa, gsb, ssa, ssb, _CPT1)

        @pl.when(c == 1)
        def _():
            _agg_run(t1, idx_v, sdx_v, rows_v, acc, gsa, gsb, ssa, ssb, _CPT1)

        plsc.subcore_barrier()

        @pl.when(c == 0)
        def _():
            pltpu.sync_copy(acc.at[pl.ds(s * zr, zr)], o0.at[pl.ds(s * zr, zr)])

        @pl.when(c == 1)
        def _():
            pltpu.sync_copy(acc.at[pl.ds(s * zr, zr)], o1.at[pl.ds(s * zr, zr)])

    return k


@functools.lru_cache(maxsize=None)
def _rgcn_quad():
    out_t = (jax.ShapeDtypeStruct((_A1, 32), jnp.float32),) * 4
    scratch = [
        pltpu.VMEM((_CPT1, 128), jnp.int32),
        pltpu.VMEM((_CPT1, 128), jnp.int32),
        pltpu.VMEM((10, 128, 32), jnp.float32),
        pltpu.VMEM_SHARED((_A1, 32), jnp.float32),
        pltpu.SemaphoreType.DMA,
        pltpu.SemaphoreType.DMA,
        pltpu.SemaphoreType.DMA,
        pltpu.SemaphoreType.DMA,
    ]

    @functools.partial(pl.kernel, mesh=_mesh(), out_type=out_t,
                       scratch_types=scratch,
                       compiler_params=pltpu.CompilerParams(
                           use_tc_tiling_on_sc=False))
    def k(q0, q1, q2, q3, gidx, sidx, zeros, o0, o1, o2, o3,
          idx_v, sdx_v, rows_v, acc, gsa, gsb, ssa, ssb):
        c = lax.axis_index("c")
        s = lax.axis_index("s")
        zr = _A1 // 16
        rows = pl.ds(s * zr, zr)
        pltpu.sync_copy(zeros, acc.at[rows])
        pltpu.sync_copy(gidx.at[pl.ds(s * _CPT1, _CPT1)], idx_v)
        pltpu.sync_copy(sidx.at[pl.ds(s * _CPT1, _CPT1)], sdx_v)
        plsc.subcore_barrier()

        @pl.when(c == 0)
        def _():
            _agg_run(q0, idx_v, sdx_v, rows_v, acc, gsa, gsb, ssa, ssb, _CPT1)

        @pl.when(c == 1)
        def _():
            _agg_run(q2, idx_v, sdx_v, rows_v, acc, gsa, gsb, ssa, ssb, _CPT1)

        plsc.subcore_barrier()

        @pl.when(c == 0)
        def _():
            pltpu.sync_copy(acc.at[rows], o0.at[rows])

        @pl.when(c == 1)
        def _():
            pltpu.sync_copy(acc.at[rows], o2.at[rows])

        pltpu.sync_copy(zeros, acc.at[rows])
        plsc.subcore_barrier()

        @pl.when(c == 0)
        def _():
            _agg_run(q1, idx_v, sdx_v, rows_v, acc, gsa, gsb, ssa, ssb, _CPT1)

        @pl.when(c == 1)
        def _():
            _agg_run(q3, idx_v, sdx_v, rows_v, acc, gsa, gsb, ssa, ssb, _CPT1)

        plsc.subcore_barrier()

        @pl.when(c == 0)
        def _():
            pltpu.sync_copy(acc.at[rows], o1.at[rows])

        @pl.when(c == 1)
        def _():
            pltpu.sync_copy(acc.at[rows], o3.at[rows])

    return k


@functools.lru_cache(maxsize=None)
def _spmm_agg(d):
    d2 = d // 2
    out_t = (jax.ShapeDtypeStruct((_A2, d2), jnp.float32),) * 2
    scratch = [
        pltpu.VMEM((_CPTS, 128), jnp.int32),
        pltpu.VMEM((_CPTS, 128), jnp.int32),
        pltpu.VMEM((_CPTS, 128), jnp.float32),
        pltpu.VMEM((10, 128, d2), jnp.float32),
        pltpu.VMEM_SHARED((_A2, d2), jnp.float32),
        pltpu.SemaphoreType.DMA,
        pltpu.SemaphoreType.DMA,
        pltpu.SemaphoreType.DMA,
        pltpu.SemaphoreType.DMA,
    ]
    ncg = d2 // 16

    @functools.partial(pl.kernel, mesh=_mesh(), out_type=out_t,
                       scratch_types=scratch,
                       compiler_params=pltpu.CompilerParams(
                           use_tc_tiling_on_sc=False))
    def k(t0, t1, gidx, sidx, w, zeros, o0, o1, idx_v, sdx_v, w_v, rows_v, acc,
          gsa, gsb, ssa, ssb):
        c = lax.axis_index("c")
        s = lax.axis_index("s")
        zr = _A2 // 16
        pltpu.sync_copy(zeros, acc.at[pl.ds(s * zr, zr)])
        pltpu.sync_copy(gidx.at[pl.ds(s * _CPTS, _CPTS)], idx_v)
        pltpu.sync_copy(sidx.at[pl.ds(s * _CPTS, _CPTS)], sdx_v)
        pltpu.sync_copy(w.at[pl.ds(s * _CPTS, _CPTS)], w_v)
        plsc.subcore_barrier()
        K = 5
        ng2 = _CPTS // (2 * K)

        def run(tbl):
            dummy = tbl.at[pl.ds(0, 128)]

            def fire(go, bufo, sem):
                for b in range(K):
                    pltpu.async_copy(tbl.at[idx_v.at[go + b]],
                                     rows_v.at[bufo + b], sem)

            def drain(sem):
                for b in range(K):
                    pltpu.make_async_copy(dummy, rows_v.at[b], sem).wait()

            def mul(go, bufo):
                for b in range(K):
                    def mgrp(g, cc, _b=b):
                        wvec = w_v[go + _b, pl.ds(g * 16, 16)]
                        for i in range(16):
                            r = g * 16 + i
                            wv = wvec[i]
                            for kk in range(ncg):
                                sl = pl.ds(kk * 16, 16)
                                rows_v[bufo + _b, r, sl] = (
                                    rows_v[bufo + _b, r, sl] * wv)
                        return cc
                    lax.fori_loop(0, 8, mgrp, 0)

            def fire_scat(go, bufo, sem):
                for b in range(K):
                    pltpu.async_copy(rows_v.at[bufo + b],
                                     acc.at[sdx_v.at[go + b]], sem, add=True)

            fire(0, 0, gsa)

            def body(g2, carry):
                g0 = 2 * g2 * K
                fire(g0 + K, K, gsb)
                drain(gsa)
                mul(g0, 0)
                fire_scat(g0, 0, ssa)
                drain(ssa)

                @pl.when(g2 < ng2 - 1)
                def _():
                    fire(g0 + 2 * K, 0, gsa)

                drain(gsb)
                mul(g0 + K, K)
                fire_scat(g0 + K, K, ssb)
                drain(ssb)
                return carry
            lax.fori_loop(0, ng2, body, 0)

        @pl.when(c == 0)
        def _():
            run(t0)

        @pl.when(c == 1)
        def _():
            run(t1)

        plsc.subcore_barrier()
        wb = _A2 // 16

        @pl.when(c == 0)
        def _():
            pltpu.sync_copy(acc.at[pl.ds(s * wb, wb)], o0.at[pl.ds(s * wb, wb)])

        @pl.when(c == 1)
        def _():
            pltpu.sync_copy(acc.at[pl.ds(s * wb, wb)], o1.at[pl.ds(s * wb, wb)])

    return k


@functools.lru_cache(maxsize=None)
def _emb_gather():
    out_t = jax.ShapeDtypeStruct((_NPAD, 32), jnp.float32)
    scratch = [
        pltpu.VMEM((3, 128), jnp.int32),
        pltpu.VMEM((128, 32), jnp.float32),
        pltpu.SemaphoreType.DMA,
    ]

    @functools.partial(pl.kernel, mesh=_mesh(), out_type=out_t,
                       scratch_types=scratch,
                       compiler_params=pltpu.CompilerParams(
                           use_tc_tiling_on_sc=False))
    def k(tbl, gidx, out, idx_v, rows_v, sem):
        c = lax.axis_index("c")
        s = lax.axis_index("s")
        w = c * 16 + s
        pltpu.sync_copy(gidx.at[w], idx_v)
        for j in range(3):
            pltpu.async_copy(tbl.at[idx_v.at[j]], rows_v, sem).wait()
            pltpu.sync_copy(rows_v, out.at[pl.ds((w * 3 + j) * 128, 128)])

    return k


def _pad_reshape(x, total, fill):
    return jnp.concatenate(
        [x, jnp.full((total - x.shape[0],), fill, x.dtype)]).reshape(-1, 128)


def kernel(user_feature, edge_index, edge_type, G_features, adj_indices,
           adj_values, mu, n_id, params, num_cluster_iter):
    N = _N
    x_ae = user_feature[:, 8:]
    cat_in = user_feature[:, :3]
    num_in = user_feature[:, 3:8]

    e0 = _dense([x_ae], [params['enc_in'][0]], params['enc_in'][1], "relu")
    e1 = _dense([e0], [params['hid_enc'][0][0]], params['hid_enc'][0][1], "relu")

    src, dst = edge_index[0], edge_index[1]
    et = edge_type.astype(jnp.int32)
    gidx1 = _pad_reshape(et * N + src, _EPAD1, 0)
    sidx1 = _pad_reshape(et * N + dst, _EPAD1, 2 * N)
    row, col = adj_indices[0], adj_indices[1]
    gidx2 = _pad_reshape(col, _EPAD2, 0)
    sidx2 = _pad_reshape(row, _EPAD2, N)
    wpad = _pad_reshape(adj_values, _EPAD2, 0.0)
    eidx = _pad_reshape(n_id.astype(jnp.int32), _NPAD, 0).reshape(32, 3, 128)

    zeros_a1 = jnp.zeros((_A1 // 16, 32), jnp.float32)
    zeros_a2 = {d: jnp.zeros((_A2 // 16, d // 2), jnp.float32) for d in (64, 32)}
    ones_t = jnp.ones((2 * N, 32), jnp.float32)

    c0, _unused = _rgcn_half()(ones_t, ones_t, gidx1, sidx1, zeros_a1)
    cnt = c0[:2 * N, 0]
    inv = (1.0 / jnp.maximum(cnt, 1.0)).reshape(2, N, 1)

    def spmm(x):
        d = x.shape[1]
        d2 = d // 2
        o0, o1 = _spmm_agg(d)(x[:, :d2], x[:, d2:], gidx2, sidx2, wpad,
                              zeros_a2[d])
        return jnp.concatenate([o0[:_N], o1[:_N]], axis=1)

    g1 = _dense([G_features], [params['gc1'][0]])
    h = jax.nn.relu(spmm(g1) + params['gc1'][1])
    h2 = _dense([h], [params['gc2'][0]])
    embeds = spmm(h2) + params['gc2'][1]
    emb_g = _emb_gather()(embeds, eidx)[:N]

    c = _dense([cat_in], [params['cat'][0]], params['cat'][1])
    nm = _dense([num_in], [params['num'][0]], params['num'][1])
    tw = _dense([x_ae], [params['tweet'][0]], params['tweet'][1])
    Wf, bf = params['feat']
    f0 = _dense([c, nm, tw], [Wf[:128], Wf[128:256], Wf[256:]], bf)

    def rgcn(xs, p):
        Wrel, Wroot, b = p
        off = [0]
        for x in xs:
            off.append(off[-1] + x.shape[1])
        t0 = _dense(xs, [Wrel[0, off[j]:off[j + 1]] for j in range(len(xs))])
        t1 = _dense(xs, [Wrel[1, off[j]:off[j + 1]] for j in range(len(xs))])
        root = _dense(xs, [Wroot[off[j]:off[j + 1]] for j in range(len(xs))], b)
        O = t0.shape[1]
        tcat = jnp.concatenate([t0, t1], axis=0)
        if O == 128:
            qs = [tcat[:, i * 32:(i + 1) * 32] for i in range(4)]
            os_ = _rgcn_quad()(qs[0], qs[1], qs[2], qs[3], gidx1, sidx1,
                               zeros_a1)
        else:
            os_ = _rgcn_half()(tcat[:, :32], tcat[:, 32:], gidx1, sidx1,
                               zeros_a1)
        acc = jnp.concatenate([o[:2 * N] for o in os_],
                              axis=1).reshape(2, N, O)
        return root + (acc * inv).sum(0)

    f1 = rgcn([f0], params['gnn_in'])
    f2 = rgcn([f1, e0, emb_g], params['hid_gnn'][0])
    f3 = rgcn([f2, e1, emb_g], params['hid_gnn'][1])
    f4 = rgcn([f3], params['gnn_out'])
    f5 = rgcn([f4], params['gnn_comm'])
    feat = jax.nn.sigmoid(f5)

    h1 = _dense([feat], [params['bd1'][0]], params['bd1'][1], "relu")
    logits = _dense([h1], [params['bd2'][0]], params['bd2'][1])
    return jax.nn.softmax(logits, axis=1)


# speedup vs baseline: 7.1968x; 1.0265x over previous
import functools

import jax
import jax.numpy as jnp
from jax import lax
from jax.experimental import pallas as pl
from jax.experimental.pallas import tpu as pltpu
from jax.experimental.pallas import tpu_sc as plsc

_N = 10000
_E = 320000
_BLK = 1000

_CPT1 = 160
_EPAD1 = _CPT1 * 16 * 128
_A1 = 20032
_CPT2 = 80
_EPAD2 = _CPT2 * 32 * 128
_A2 = 10080
_CPTS = 160
_NPAD = 12288

def _mesh():
    return plsc.VectorSubcoreMesh(core_axis_name="c", subcore_axis_name="s")


def _dense_body(nx, has_bias, act, *refs):
    xs = refs[:nx]
    ws = refs[nx:2 * nx]
    i = 2 * nx
    acc = xs[0][...] @ ws[0][...]
    for j in range(1, nx):
        acc = acc + xs[j][...] @ ws[j][...]
    if has_bias:
        acc = acc + refs[i][...]
        i += 1
    if act == "relu":
        acc = jnp.maximum(acc, 0.0)
    elif act == "sigmoid":
        acc = jax.nn.sigmoid(acc)
    refs[-1][...] = acc


def _dense(xs, Ws, b=None, act=None):
    n = xs[0].shape[0]
    O = Ws[0].shape[1]
    nx = len(xs)
    in_specs = [pl.BlockSpec((_BLK, x.shape[1]), lambda i: (i, 0)) for x in xs]
    in_specs += [pl.BlockSpec(W.shape, lambda i: (0, 0)) for W in Ws]
    args = list(xs) + list(Ws)
    if b is not None:
        in_specs.append(pl.BlockSpec((1, O), lambda i: (0, 0)))
        args.append(b.reshape(1, O))
    return pl.pallas_call(
        functools.partial(_dense_body, nx, b is not None, act),
        grid=(n // _BLK,),
        in_specs=in_specs,
        out_specs=pl.BlockSpec((_BLK, O), lambda i: (i, 0)),
        out_shape=jax.ShapeDtypeStruct((n, O), jnp.float32),
    )(*args)


def _agg_run(tbl, idx_v, sdx_v, rows_v, acc, gsa, gsb, ssa, ssb, cpt):
    K = 5
    ng2 = cpt // (2 * K)
    dummy = tbl.at[pl.ds(0, 128)]

    def fire(go, bufo, sem):
        for b in range(K):
            pltpu.async_copy(tbl.at[idx_v.at[go + b]], rows_v.at[bufo + b],
                             sem)

    def drain(sem):
        for b in range(K):
            pltpu.make_async_copy(dummy, rows_v.at[b], sem).wait()

    def fire_scat(go, bufo, sem):
        for b in range(K):
            pltpu.async_copy(rows_v.at[bufo + b], acc.at[sdx_v.at[go + b]],
                             sem, add=True)

    fire(0, 0, gsa)

    def body(g2, carry):
        g0 = 2 * g2 * K
        fire(g0 + K, K, gsb)
        drain(gsa)
        fire_scat(g0, 0, ssa)
        drain(ssa)

        @pl.when(g2 < ng2 - 1)
        def _():
            fire(g0 + 2 * K, 0, gsa)

        drain(gsb)
        fire_scat(g0 + K, K, ssb)
        drain(ssb)
        return carry
    lax.fori_loop(0, ng2, body, 0)


@functools.lru_cache(maxsize=None)
def _rgcn_half():
    out_t = (jax.ShapeDtypeStruct((_A1, 32), jnp.float32),) * 2
    scratch = [
        pltpu.VMEM((_CPT1, 128), jnp.int32),
        pltpu.VMEM((_CPT1, 128), jnp.int32),
        pltpu.VMEM((10, 128, 32), jnp.float32),
        pltpu.VMEM_SHARED((_A1, 32), jnp.float32),
        pltpu.SemaphoreType.DMA,
        pltpu.SemaphoreType.DMA,
        pltpu.SemaphoreType.DMA,
        pltpu.SemaphoreType.DMA,
    ]

    @functools.partial(pl.kernel, mesh=_mesh(), out_type=out_t,
                       scratch_types=scratch,
                       compiler_params=pltpu.CompilerParams(
                           use_tc_tiling_on_sc=False))
    def k(t0, t1, gidx, sidx, zeros, o0, o1, idx_v, sdx_v, rows_v, acc,
          gsa, gsb, ssa, ssb):
        c = lax.axis_index("c")
        s = lax.axis_index("s")
        zr = _A1 // 16
        pltpu.sync_copy(zeros, acc.at[pl.ds(s * zr, zr)])
        pltpu.sync_copy(gidx.at[pl.ds(s * _CPT1, _CPT1)], idx_v)
        pltpu.sync_copy(sidx.at[pl.ds(s * _CPT1, _CPT1)], sdx_v)
        plsc.subcore_barrier()

        @pl.when(c == 0)
        def _():
            _agg_run(t0, idx_v, sdx_v, rows_v, acc, gsa, gsb, ssa, ssb, _CPT1)

        @pl.when(c == 1)
        def _():
            _agg_run(t1, idx_v, sdx_v, rows_v, acc, gsa, gsb, ssa, ssb, _CPT1)

        plsc.subcore_barrier()

        @pl.when(c == 0)
        def _():
            pltpu.sync_copy(acc.at[pl.ds(s * zr, zr)], o0.at[pl.ds(s * zr, zr)])

        @pl.when(c == 1)
        def _():
            pltpu.sync_copy(acc.at[pl.ds(s * zr, zr)], o1.at[pl.ds(s * zr, zr)])

    return k


@functools.lru_cache(maxsize=None)
def _rgcn_quad():
    out_t = (jax.ShapeDtypeStruct((_A1, 32), jnp.float32),) * 4
    scratch = [
        pltpu.VMEM((_CPT1, 128), jnp.int32),
        pltpu.VMEM((_CPT1, 128), jnp.int32),
        pltpu.VMEM((10, 128, 32), jnp.float32),
        pltpu.VMEM_SHARED((_A1, 32), jnp.float32),
        pltpu.SemaphoreType.DMA,
        pltpu.SemaphoreType.DMA,
        pltpu.SemaphoreType.DMA,
        pltpu.SemaphoreType.DMA,
    ]

    @functools.partial(pl.kernel, mesh=_mesh(), out_type=out_t,
                       scratch_types=scratch,
                       compiler_params=pltpu.CompilerParams(
                           use_tc_tiling_on_sc=False))
    def k(q0, q1, q2, q3, gidx, sidx, zeros, o0, o1, o2, o3,
          idx_v, sdx_v, rows_v, acc, gsa, gsb, ssa, ssb):
        c = lax.axis_index("c")
        s = lax.axis_index("s")
        zr = _A1 // 16
        rows = pl.ds(s * zr, zr)
        pltpu.sync_copy(zeros, acc.at[rows])
        pltpu.sync_copy(gidx.at[pl.ds(s * _CPT1, _CPT1)], idx_v)
        pltpu.sync_copy(sidx.at[pl.ds(s * _CPT1, _CPT1)], sdx_v)
        plsc.subcore_barrier()

        @pl.when(c == 0)
        def _():
            _agg_run(q0, idx_v, sdx_v, rows_v, acc, gsa, gsb, ssa, ssb, _CPT1)

        @pl.when(c == 1)
        def _():
            _agg_run(q2, idx_v, sdx_v, rows_v, acc, gsa, gsb, ssa, ssb, _CPT1)

        plsc.subcore_barrier()

        @pl.when(c == 0)
        def _():
            pltpu.sync_copy(acc.at[rows], o0.at[rows])

        @pl.when(c == 1)
        def _():
            pltpu.sync_copy(acc.at[rows], o2.at[rows])

        pltpu.sync_copy(zeros, acc.at[rows])
        plsc.subcore_barrier()

        @pl.when(c == 0)
        def _():
            _agg_run(q1, idx_v, sdx_v, rows_v, acc, gsa, gsb, ssa, ssb, _CPT1)

        @pl.when(c == 1)
        def _():
            _agg_run(q3, idx_v, sdx_v, rows_v, acc, gsa, gsb, ssa, ssb, _CPT1)

        plsc.subcore_barrier()

        @pl.when(c == 0)
        def _():
            pltpu.sync_copy(acc.at[rows], o1.at[rows])

        @pl.when(c == 1)
        def _():
            pltpu.sync_copy(acc.at[rows], o3.at[rows])

    return k


@functools.lru_cache(maxsize=None)
def _spmm_agg(d):
    d2 = d // 2
    out_t = (jax.ShapeDtypeStruct((_A2, d2), jnp.float32),) * 2
    scratch = [
        pltpu.VMEM((_CPTS, 128), jnp.int32),
        pltpu.VMEM((_CPTS, 128), jnp.int32),
        pltpu.VMEM((_CPTS, 128), jnp.float32),
        pltpu.VMEM((10, 128, d2), jnp.float32),
        pltpu.VMEM_SHARED((_A2, d2), jnp.float32),
        pltpu.SemaphoreType.DMA,
        pltpu.SemaphoreType.DMA,
        pltpu.SemaphoreType.DMA,
        pltpu.SemaphoreType.DMA,
    ]
    ncg = d2 // 16

    @functools.partial(pl.kernel, mesh=_mesh(), out_type=out_t,
                       scratch_types=scratch,
                       compiler_params=pltpu.CompilerParams(
                           use_tc_tiling_on_sc=False))
    def k(t0, t1, gidx, sidx, w, zeros, o0, o1, idx_v, sdx_v, w_v, rows_v, acc,
          gsa, gsb, ssa, ssb):
        c = lax.axis_index("c")
        s = lax.axis_index("s")
        zr = _A2 // 16
        pltpu.sync_copy(zeros, acc.at[pl.ds(s * zr, zr)])
        pltpu.sync_copy(gidx.at[pl.ds(s * _CPTS, _CPTS)], idx_v)
        pltpu.sync_copy(sidx.at[pl.ds(s * _CPTS, _CPTS)], sdx_v)
        pltpu.sync_copy(w.at[pl.ds(s * _CPTS, _CPTS)], w_v)
        plsc.subcore_barrier()
        K = 5
        ng2 = _CPTS // (2 * K)

        def run(tbl):
            dummy = tbl.at[pl.ds(0, 128)]

            def fire(go, bufo, sem):
                for b in range(K):
                    pltpu.async_copy(tbl.at[idx_v.at[go + b]],
                                     rows_v.at[bufo + b], sem)

            def drain(sem):
                for b in range(K):
                    pltpu.make_async_copy(dummy, rows_v.at[b], sem).wait()

            def mul(go, bufo):
                for b in range(K):
                    def mgrp(g, cc, _b=b):
                        wvec = w_v[go + _b, pl.ds(g * 16, 16)]
                        for i in range(16):
                            r = g * 16 + i
                            wv = wvec[i]
                            for kk in range(ncg):
                                sl = pl.ds(kk * 16, 16)
                                rows_v[bufo + _b, r, sl] = (
                                    rows_v[bufo + _b, r, sl] * wv)
                        return cc
                    lax.fori_loop(0, 8, mgrp, 0)

            def fire_scat(go, bufo, sem):
                for b in range(K):
                    pltpu.async_copy(rows_v.at[bufo + b],
                                     acc.at[sdx_v.at[go + b]], sem, add=True)

            fire(0, 0, gsa)

            def body(g2, carry):
                g0 = 2 * g2 * K
                fire(g0 + K, K, gsb)
                drain(gsa)
                mul(g0, 0)
                fire_scat(g0, 0, ssa)
                drain(ssa)

                @pl.when(g2 < ng2 - 1)
                def _():
                    fire(g0 + 2 * K, 0, gsa)

                drain(gsb)
                mul(g0 + K, K)
                fire_scat(g0 + K, K, ssb)
                drain(ssb)
                return carry
            lax.fori_loop(0, ng2, body, 0)

        @pl.when(c == 0)
        def _():
            run(t0)

        @pl.when(c == 1)
        def _():
            run(t1)

        plsc.subcore_barrier()
        wb = _A2 // 16

        @pl.when(c == 0)
        def _():
            pltpu.sync_copy(acc.at[pl.ds(s * wb, wb)], o0.at[pl.ds(s * wb, wb)])

        @pl.when(c == 1)
        def _():
            pltpu.sync_copy(acc.at[pl.ds(s * wb, wb)], o1.at[pl.ds(s * wb, wb)])

    return k


@functools.lru_cache(maxsize=None)
def _cnt_kernel():
    out_t = (jax.ShapeDtypeStruct((_A1, 16), jnp.float32),) * 2
    scratch = [
        pltpu.VMEM((_CPT2, 128), jnp.int32),
        pltpu.VMEM((128, 16), jnp.float32),
        pltpu.VMEM_SHARED((_A1, 16), jnp.float32),
        pltpu.SemaphoreType.DMA,
    ]

    @functools.partial(pl.kernel, mesh=_mesh(), out_type=out_t,
                       scratch_types=scratch,
                       compiler_params=pltpu.CompilerParams(
                           use_tc_tiling_on_sc=False))
    def k(sidx, ones, zeros, o0, o1, sdx_v, ones_v, acc, csem):
        c = lax.axis_index("c")
        s = lax.axis_index("s")
        zr = _A1 // 16
        pltpu.sync_copy(zeros, acc.at[pl.ds(s * zr, zr)])
        base = (c * 16 + s) * _CPT2
        pltpu.sync_copy(sidx.at[pl.ds(base, _CPT2)], sdx_v)
        pltpu.sync_copy(ones, ones_v)
        plsc.subcore_barrier()

        def body(j, carry):
            pltpu.async_copy(ones_v, acc.at[sdx_v.at[j]], csem, add=True)
            return carry
        lax.fori_loop(0, _CPT2, body, 0)

        def dbody(j, carry):
            pltpu.make_async_copy(ones, ones_v, csem).wait()
            return carry
        lax.fori_loop(0, _CPT2, dbody, 0)

        plsc.subcore_barrier()

        @pl.when(c == 0)
        def _():
            pltpu.sync_copy(acc.at[pl.ds(s * zr, zr)], o0.at[pl.ds(s * zr, zr)])

        @pl.when(c == 1)
        def _():
            pltpu.sync_copy(acc.at[pl.ds(s * zr, zr)], o1.at[pl.ds(s * zr, zr)])

    return k


@functools.lru_cache(maxsize=None)
def _emb_gather():
    out_t = jax.ShapeDtypeStruct((_NPAD, 32), jnp.float32)
    scratch = [
        pltpu.VMEM((3, 128), jnp.int32),
        pltpu.VMEM((128, 32), jnp.float32),
        pltpu.SemaphoreType.DMA,
    ]

    @functools.partial(pl.kernel, mesh=_mesh(), out_type=out_t,
                       scratch_types=scratch,
                       compiler_params=pltpu.CompilerParams(
                           use_tc_tiling_on_sc=False))
    def k(tbl, gidx, out, idx_v, rows_v, sem):
        c = lax.axis_index("c")
        s = lax.axis_index("s")
        w = c * 16 + s
        pltpu.sync_copy(gidx.at[w], idx_v)
        for j in range(3):
            pltpu.async_copy(tbl.at[idx_v.at[j]], rows_v, sem).wait()
            pltpu.sync_copy(rows_v, out.at[pl.ds((w * 3 + j) * 128, 128)])

    return k


def _pad_reshape(x, total, fill):
    return jnp.concatenate(
        [x, jnp.full((total - x.shape[0],), fill, x.dtype)]).reshape(-1, 128)


def kernel(user_feature, edge_index, edge_type, G_features, adj_indices,
           adj_values, mu, n_id, params, num_cluster_iter):
    N = _N
    x_ae = user_feature[:, 8:]
    cat_in = user_feature[:, :3]
    num_in = user_feature[:, 3:8]

    e0 = _dense([x_ae], [params['enc_in'][0]], params['enc_in'][1], "relu")
    e1 = _dense([e0], [params['hid_enc'][0][0]], params['hid_enc'][0][1], "relu")

    src, dst = edge_index[0], edge_index[1]
    et = edge_type.astype(jnp.int32)
    gidx1 = _pad_reshape(et * N + src, _EPAD1, 0)
    sidx1 = _pad_reshape(et * N + dst, _EPAD1, 2 * N)
    row, col = adj_indices[0], adj_indices[1]
    gidx2 = _pad_reshape(col, _EPAD2, 0)
    sidx2 = _pad_reshape(row, _EPAD2, N)
    wpad = _pad_reshape(adj_values, _EPAD2, 0.0)
    csidx = _pad_reshape(et * N + dst, _EPAD2, 2 * N)
    eidx = _pad_reshape(n_id.astype(jnp.int32), _NPAD, 0).reshape(32, 3, 128)

    zeros_a1 = jnp.zeros((_A1 // 16, 32), jnp.float32)
    zeros_a2 = {d: jnp.zeros((_A2 // 16, d // 2), jnp.float32) for d in (64, 32)}
    zeros_c = jnp.zeros((_A1 // 16, 16), jnp.float32)
    ones_c = jnp.ones((128, 16), jnp.float32)

    c0, c1 = _cnt_kernel()(csidx, ones_c, zeros_c)
    cnt = (c0 + c1)[:2 * N, 0]
    inv = (1.0 / jnp.maximum(cnt, 1.0)).reshape(2, N, 1)

    def spmm(x):
        d = x.shape[1]
        d2 = d // 2
        o0, o1 = _spmm_agg(d)(x[:, :d2], x[:, d2:], gidx2, sidx2, wpad,
                              zeros_a2[d])
        return jnp.concatenate([o0[:_N], o1[:_N]], axis=1)

    g1 = _dense([G_features], [params['gc1'][0]])
    h = jax.nn.relu(spmm(g1) + params['gc1'][1])
    h2 = _dense([h], [params['gc2'][0]])
    embeds = spmm(h2) + params['gc2'][1]
    emb_g = _emb_gather()(embeds, eidx)[:N]

    c = _dense([cat_in], [params['cat'][0]], params['cat'][1])
    nm = _dense([num_in], [params['num'][0]], params['num'][1])
    tw = _dense([x_ae], [params['tweet'][0]], params['tweet'][1])
    Wf, bf = params['feat']
    f0 = _dense([c, nm, tw], [Wf[:128], Wf[128:256], Wf[256:]], bf)

    def rgcn(xs, p):
        Wrel, Wroot, b = p
        off = [0]
        for x in xs:
            off.append(off[-1] + x.shape[1])
        t0 = _dense(xs, [Wrel[0, off[j]:off[j + 1]] for j in range(len(xs))])
        t1 = _dense(xs, [Wrel[1, off[j]:off[j + 1]] for j in range(len(xs))])
        root = _dense(xs, [Wroot[off[j]:off[j + 1]] for j in range(len(xs))], b)
        O = t0.shape[1]
        tcat = jnp.concatenate([t0, t1], axis=0)
        if O == 128:
            qs = [tcat[:, i * 32:(i + 1) * 32] for i in range(4)]
            os_ = _rgcn_quad()(qs[0], qs[1], qs[2], qs[3], gidx1, sidx1,
                               zeros_a1)
        else:
            os_ = _rgcn_half()(tcat[:, :32], tcat[:, 32:], gidx1, sidx1,
                               zeros_a1)
        acc = jnp.concatenate([o[:2 * N] for o in os_],
                              axis=1).reshape(2, N, O)
        return root + (acc * inv).sum(0)

    f1 = rgcn([f0], params['gnn_in'])
    f2 = rgcn([f1, e0, emb_g], params['hid_gnn'][0])
    f3 = rgcn([f2, e1, emb_g], params['hid_gnn'][1])
    f4 = rgcn([f3], params['gnn_out'])
    f5 = rgcn([f4], params['gnn_comm'])
    feat = jax.nn.sigmoid(f5)

    h1 = _dense([feat], [params['bd1'][0]], params['bd1'][1], "relu")
    logits = _dense([h1], [params['bd2'][0]], params['bd2'][1])
    return jax.nn.softmax(logits, axis=1)
